# baseline (speedup 1.0000x reference)
import jax
import jax.numpy as jnp
from jax.experimental import pallas as pl
from jax.experimental.pallas import tpu as pltpu

SEQ = 9
DM = 1024
ROWS = 2048
BLK = 1024


def _body(x_ref, w_ref, o_ref):
    w = w_ref[0]
    ss = jnp.sum(w * w)
    norm = jnp.sqrt(ss)
    scale = jnp.where(norm > 1.0, 1.0 / (norm + 1e-7), 1.0)
    pe = (w * scale)[:, None, None, :]
    o_ref[...] = x_ref[...] + pe


def kernel(x, W):
    B = x.shape[0]
    grid = (B, SEQ, ROWS // BLK)
    W3 = W.reshape(SEQ, 1, DM)
    return pl.pallas_call(
        _body,
        grid=grid,
        in_specs=[
            pl.BlockSpec((1, 1, BLK, DM), lambda b, s, c: (b, s, c, 0)),
            pl.BlockSpec((1, 1, DM), lambda b, s, c: (s, 0, 0)),
        ],
        out_specs=pl.BlockSpec((1, 1, BLK, DM), lambda b, s, c: (b, s, c, 0)),
        out_shape=jax.ShapeDtypeStruct(x.shape, x.dtype),
        compiler_params=pltpu.CompilerParams(
            dimension_semantics=("parallel", "parallel", "parallel"),
        ),
    )(x, W3)


# speedup vs baseline: 1.0255x; 1.0255x over previous
import jax
import jax.numpy as jnp
from jax.experimental import pallas as pl
from jax.experimental.pallas import tpu as pltpu

SEQ = 9
DM = 1024
ROWS = 2048
PAD = 16


def _body(x_ref, w_ref, o_ref, pe_ref):
    b = pl.program_id(0)
    s = pl.program_id(1)

    @pl.when(jnp.logical_and(b == 0, s == 0))
    def _init():
        w = w_ref[:, 0, :]
        ss = jnp.sum(w * w, axis=-1, keepdims=True)
        norm = jnp.sqrt(ss)
        scale = jnp.where(norm > 1.0, 1.0 / (norm + 1e-7), 1.0)
        pe_ref[...] = w * scale

    pe = pe_ref[pl.ds(s, 1), :]
    o_ref[...] = x_ref[...] + pe[None, :, None, :]


def kernel(x, W):
    B = x.shape[0]
    Wp = jnp.zeros((PAD, 1, DM), W.dtype).at[:SEQ, 0, :].set(W)
    return pl.pallas_call(
        _body,
        grid=(B, SEQ),
        in_specs=[
            pl.BlockSpec((1, 1, ROWS, DM), lambda b, s: (b, s, 0, 0)),
            pl.BlockSpec((PAD, 1, DM), lambda b, s: (0, 0, 0)),
        ],
        out_specs=pl.BlockSpec((1, 1, ROWS, DM), lambda b, s: (b, s, 0, 0)),
        out_shape=jax.ShapeDtypeStruct(x.shape, x.dtype),
        scratch_shapes=[pltpu.VMEM((PAD, DM), jnp.float32)],
        compiler_params=pltpu.CompilerParams(
            dimension_semantics=("arbitrary", "arbitrary"),
        ),
    )(x, Wp)
